# baseline (speedup 1.0000x reference)
import functools

import jax
import jax.numpy as jnp
from jax import lax
from jax.experimental import pallas as pl
from jax.experimental.pallas import tpu as pltpu
from jax.experimental.pallas import tpu_sc as plsc

B = 16384
D_GMF = 32
D_MLP = 128
NC = 2
NS = 16
NW = NC * NS
BPW = B // NW
B_GMF_ROWS = 250000
CH = 128
NCH = BPW // CH


def _gather_pair(u_h, i_h, out_u, out_i, uidx, iidx, bu, bi, sem, base):
    nbuf = 2
    copies = [None] * (2 * NCH)

    def fire(j):
        copies[2 * j] = pltpu.async_copy(u_h.at[uidx.at[j]], bu.at[j % nbuf],
                                         sem)
        copies[2 * j + 1] = pltpu.async_copy(i_h.at[iidx.at[j]],
                                             bi.at[j % nbuf], sem)

    for j in range(nbuf):
        fire(j)
    for j in range(NCH):
        r0 = base + j * CH
        copies[2 * j].wait()
        pltpu.sync_copy(bu.at[j % nbuf], out_u.at[pl.ds(r0, CH)])
        copies[2 * j + 1].wait()
        pltpu.sync_copy(bi.at[j % nbuf], out_i.at[pl.ds(r0, CH)])
        if j + nbuf < NCH:
            fire(j + nbuf)


@functools.cache
def _make_sc_mlp_gather():
    mesh = plsc.VectorSubcoreMesh(core_axis_name="c", subcore_axis_name="s")

    @functools.partial(
        pl.kernel,
        mesh=mesh,
        out_type=(
            jax.ShapeDtypeStruct((B, D_MLP), jnp.float32),
            jax.ShapeDtypeStruct((B, D_MLP), jnp.float32),
        ),
        scratch_types=[
            pltpu.VMEM((NCH, CH), jnp.int32),
            pltpu.VMEM((NCH, CH), jnp.int32),
            pltpu.VMEM((2, CH, D_MLP), jnp.float32),
            pltpu.VMEM((2, CH, D_MLP), jnp.float32),
            pltpu.SemaphoreType.DMA,
        ],
    )
    def _sc_mlp(user_h, item_h, um_h, im_h, out_um, out_im,
                uidx, iidx, bum, bim, sem):
        wid = lax.axis_index("s") * NC + lax.axis_index("c")
        base = wid * BPW
        pltpu.sync_copy(user_h.at[wid], uidx)
        pltpu.sync_copy(item_h.at[wid], iidx)
        _gather_pair(um_h, im_h, out_um, out_im, uidx, iidx, bum, bim, sem,
                     base)

    return _sc_mlp


@functools.cache
def _make_sc_gmf_gather():
    mesh = plsc.VectorSubcoreMesh(core_axis_name="c", subcore_axis_name="s")

    @functools.partial(
        pl.kernel,
        mesh=mesh,
        out_type=(
            jax.ShapeDtypeStruct((B, D_MLP), jnp.float32),
            jax.ShapeDtypeStruct((B, D_MLP), jnp.float32),
        ),
        scratch_types=[
            pltpu.VMEM((NCH, CH), jnp.int32),
            pltpu.VMEM((NCH, CH), jnp.int32),
            pltpu.VMEM((2, CH, D_MLP), jnp.float32),
            pltpu.VMEM((2, CH, D_MLP), jnp.float32),
            pltpu.SemaphoreType.DMA,
        ],
    )
    def _sc_gmf(user_h, item_h, ug_h, ig_h, out_ug, out_ig,
                uidx, iidx, bug, big, sem):
        wid = lax.axis_index("s") * NC + lax.axis_index("c")
        base = wid * BPW
        pltpu.sync_copy(user_h.at[wid], uidx)
        pltpu.sync_copy(item_h.at[wid], iidx)
        _gather_pair(ug_h, ig_h, out_ug, out_ig, uidx, iidx, bug, big, sem,
                     base)

    return _sc_gmf


PACK_CHK = 4096
PACK_SHIFT = 12
PACK_Q = PACK_CHK // 4
PACK_GRID = -(-1000000 // PACK_CHK)


def _pack_body(tu, ti, ou, oi):
    q = PACK_CHK // 4
    eye = jax.lax.broadcasted_iota(jnp.int32, (D_GMF, D_GMF), 0) == \
        jax.lax.broadcasted_iota(jnp.int32, (D_GMF, D_GMF), 1)
    eye = eye.astype(jnp.float32)
    for t_ref, o_ref in ((tu, ou), (ti, oi)):
        for k in range(4):
            xk = t_ref[:, q * k:q * (k + 1)]
            yk = jax.lax.dot_general(
                xk, eye, (((0,), (0,)), ((), ())),
                preferred_element_type=jnp.float32)
            o_ref[:, 32 * k:32 * k + 32] = yk

    @pl.when(pl.program_id(0) == PACK_GRID - 1)
    def _():
        ou[:, D_GMF:] = jnp.zeros((q, D_MLP - D_GMF), jnp.float32)
        oi[:, D_GMF:] = jnp.zeros((q, D_MLP - D_GMF), jnp.float32)


_pack_call = pl.pallas_call(
    _pack_body,
    grid=(PACK_GRID,),
    in_specs=[
        pl.BlockSpec((D_GMF, PACK_CHK), lambda i: (0, i)),
        pl.BlockSpec((D_GMF, PACK_CHK), lambda i: (0, i)),
    ],
    out_specs=[
        pl.BlockSpec((PACK_CHK // 4, D_MLP), lambda i: (i, 0)),
        pl.BlockSpec((PACK_CHK // 4, D_MLP), lambda i: (i, 0)),
    ],
    out_shape=[
        jax.ShapeDtypeStruct((PACK_GRID * PACK_CHK // 4, D_MLP), jnp.float32),
        jax.ShapeDtypeStruct((PACK_GRID * PACK_CHK // 4, D_MLP), jnp.float32),
    ],
    compiler_params=pltpu.CompilerParams(fuse_transposed_lhs_in_matmul=True),
)


BLK = 1024
NB = B // BLK


def _tc_body(ug, ig, um, im, ohu, ohi, w1a, w1b, b1r, w2, b2r, w3, b3r, wpg,
             wph, bpr, out):
    ug128, ig128 = ug[...], ig[...]
    g_u = sum(ohu[...][:, k:k + 1] * ug128[:, 32 * k:32 * k + 32]
              for k in range(4))
    g_i = sum(ohi[...][:, k:k + 1] * ig128[:, 32 * k:32 * k + 32]
              for k in range(4))
    g = g_u * g_i
    h = jnp.dot(um[...], w1a[...], preferred_element_type=jnp.float32)
    h = h + jnp.dot(im[...], w1b[...], preferred_element_type=jnp.float32)
    h = jnp.maximum(h + b1r[...], 0.0)
    h = jnp.maximum(
        jnp.dot(h, w2[...], preferred_element_type=jnp.float32) + b2r[...], 0.0)
    h = jnp.maximum(
        jnp.dot(h, w3[...], preferred_element_type=jnp.float32) + b3r[...], 0.0)
    p = jnp.dot(g, wpg[...], preferred_element_type=jnp.float32)
    p = p + jnp.dot(h, wph[...], preferred_element_type=jnp.float32)
    out[...] = p + bpr[...]


def _full(shape):
    return pl.BlockSpec(shape, lambda i: (0, 0))


_tc_call = pl.pallas_call(
    _tc_body,
    grid=(NB,),
    in_specs=[
        pl.BlockSpec((BLK, D_MLP), lambda i: (i, 0)),
        pl.BlockSpec((BLK, D_MLP), lambda i: (i, 0)),
        pl.BlockSpec((BLK, D_MLP), lambda i: (i, 0)),
        pl.BlockSpec((BLK, D_MLP), lambda i: (i, 0)),
        pl.BlockSpec((BLK, 4), lambda i: (i, 0)),
        pl.BlockSpec((BLK, 4), lambda i: (i, 0)),
        _full((128, 128)),
        _full((128, 128)),
        _full((1, 128)),
        _full((128, 64)),
        _full((1, 64)),
        _full((64, 32)),
        _full((1, 32)),
        _full((32, 8)),
        _full((32, 8)),
        _full((1, 8)),
    ],
    out_specs=pl.BlockSpec((BLK, 8), lambda i: (i, 0)),
    out_shape=jax.ShapeDtypeStruct((B, 8), jnp.float32),
)


def kernel(user, item, embed_user_gmf, embed_item_gmf, embed_user_mlp,
           embed_item_mlp, W1, b1, W2, b2, W3, b3, W_pred, b_pred):
    u3 = user.reshape(NW, NCH, CH)
    i3 = item.reshape(NW, NCH, CH)
    g_um, g_im = _make_sc_mlp_gather()(
        u3, i3, embed_user_mlp, embed_item_mlp)
    q = PACK_Q
    u4 = ((user >> PACK_SHIFT) * q + (user & (q - 1))).reshape(NW, NCH, CH)
    i4 = ((item >> PACK_SHIFT) * q + (item & (q - 1))).reshape(NW, NCH, CH)
    pk_u, pk_i = _pack_call(embed_user_gmf.T, embed_item_gmf.T)
    g_ug, g_ig = _make_sc_gmf_gather()(u4, i4, pk_u, pk_i)
    lanes = jnp.arange(4, dtype=jnp.int32)[None, :]
    ohu = (((user & (PACK_CHK - 1)) // q)[:, None] == lanes).astype(
        jnp.float32)
    ohi = (((item & (PACK_CHK - 1)) // q)[:, None] == lanes).astype(
        jnp.float32)
    w1t = W1.T
    w1a, w1b = w1t[:D_MLP], w1t[D_MLP:]
    wpt = W_pred.T
    wpg = jnp.broadcast_to(wpt[:D_GMF], (D_GMF, 8))
    wph = jnp.broadcast_to(wpt[D_GMF:], (D_GMF, 8))
    bpr = jnp.broadcast_to(b_pred.reshape(1, 1), (1, 8))
    p8 = _tc_call(g_ug, g_ig, g_um, g_im, ohu, ohi, w1a, w1b,
                  b1.reshape(1, -1), W2.T, b2.reshape(1, -1), W3.T,
                  b3.reshape(1, -1), wpg, wph, bpr)
    return p8[:, 0]


# speedup vs baseline: 1.0266x; 1.0266x over previous
import functools

import jax
import jax.numpy as jnp
from jax import lax
from jax.experimental import pallas as pl
from jax.experimental.pallas import tpu as pltpu
from jax.experimental.pallas import tpu_sc as plsc

B = 16384
D_GMF = 32
D_MLP = 128
NC = 2
NS = 16
NW = NC * NS
BPW = B // NW
B_GMF_ROWS = 250000
CH = 128
NCH = BPW // CH


def _gather_pair(u_h, i_h, out_u, out_i, uidx, iidx, bu, bi, sem, base):
    nbuf = 2
    copies = [None] * (2 * NCH)

    def fire(j):
        copies[2 * j] = pltpu.async_copy(u_h.at[uidx.at[j]], bu.at[j % nbuf],
                                         sem)
        copies[2 * j + 1] = pltpu.async_copy(i_h.at[iidx.at[j]],
                                             bi.at[j % nbuf], sem)

    for j in range(nbuf):
        fire(j)
    for j in range(NCH):
        r0 = base + j * CH
        copies[2 * j].wait()
        pltpu.sync_copy(bu.at[j % nbuf], out_u.at[pl.ds(r0, CH)])
        copies[2 * j + 1].wait()
        pltpu.sync_copy(bi.at[j % nbuf], out_i.at[pl.ds(r0, CH)])
        if j + nbuf < NCH:
            fire(j + nbuf)


@functools.cache
def _make_sc_mlp_gather():
    mesh = plsc.VectorSubcoreMesh(core_axis_name="c", subcore_axis_name="s")

    @functools.partial(
        pl.kernel,
        mesh=mesh,
        out_type=(
            jax.ShapeDtypeStruct((B, D_MLP), jnp.float32),
            jax.ShapeDtypeStruct((B, D_MLP), jnp.float32),
        ),
        scratch_types=[
            pltpu.VMEM((NCH, CH), jnp.int32),
            pltpu.VMEM((NCH, CH), jnp.int32),
            pltpu.VMEM((2, CH, D_MLP), jnp.float32),
            pltpu.VMEM((2, CH, D_MLP), jnp.float32),
            pltpu.SemaphoreType.DMA,
        ],
    )
    def _sc_mlp(user_h, item_h, um_h, im_h, out_um, out_im,
                uidx, iidx, bum, bim, sem):
        wid = lax.axis_index("s") * NC + lax.axis_index("c")
        base = wid * BPW
        pltpu.sync_copy(user_h.at[wid], uidx)
        pltpu.sync_copy(item_h.at[wid], iidx)
        _gather_pair(um_h, im_h, out_um, out_im, uidx, iidx, bum, bim, sem,
                     base)

    return _sc_mlp


@functools.cache
def _make_sc_gmf_gather():
    mesh = plsc.VectorSubcoreMesh(core_axis_name="c", subcore_axis_name="s")

    @functools.partial(
        pl.kernel,
        mesh=mesh,
        out_type=(
            jax.ShapeDtypeStruct((B, D_MLP), jnp.float32),
            jax.ShapeDtypeStruct((B, D_MLP), jnp.float32),
        ),
        scratch_types=[
            pltpu.VMEM((NCH, CH), jnp.int32),
            pltpu.VMEM((NCH, CH), jnp.int32),
            pltpu.VMEM((2, CH, D_MLP), jnp.float32),
            pltpu.VMEM((2, CH, D_MLP), jnp.float32),
            pltpu.SemaphoreType.DMA,
        ],
    )
    def _sc_gmf(user_h, item_h, ug_h, ig_h, out_ug, out_ig,
                uidx, iidx, bug, big, sem):
        wid = lax.axis_index("s") * NC + lax.axis_index("c")
        base = wid * BPW
        pltpu.sync_copy(user_h.at[wid], uidx)
        pltpu.sync_copy(item_h.at[wid], iidx)
        _gather_pair(ug_h, ig_h, out_ug, out_ig, uidx, iidx, bug, big, sem,
                     base)

    return _sc_gmf


PACK_CHK = 8192
PACK_SHIFT = 13
PACK_Q = PACK_CHK // 4
PACK_GRID = -(-1000000 // PACK_CHK)


def _pack_body(tu, ti, ou, oi):
    q = PACK_CHK // 4
    eye = jax.lax.broadcasted_iota(jnp.int32, (D_GMF, D_GMF), 0) == \
        jax.lax.broadcasted_iota(jnp.int32, (D_GMF, D_GMF), 1)
    eye = eye.astype(jnp.float32)
    for t_ref, o_ref in ((tu, ou), (ti, oi)):
        for k in range(4):
            xk = t_ref[:, q * k:q * (k + 1)]
            yk = jax.lax.dot_general(
                xk, eye, (((0,), (0,)), ((), ())),
                preferred_element_type=jnp.float32)
            o_ref[:, 32 * k:32 * k + 32] = yk

    @pl.when(pl.program_id(0) == PACK_GRID - 1)
    def _():
        ou[:, D_GMF:] = jnp.zeros((q, D_MLP - D_GMF), jnp.float32)
        oi[:, D_GMF:] = jnp.zeros((q, D_MLP - D_GMF), jnp.float32)


_pack_call = pl.pallas_call(
    _pack_body,
    grid=(PACK_GRID,),
    in_specs=[
        pl.BlockSpec((D_GMF, PACK_CHK), lambda i: (0, i)),
        pl.BlockSpec((D_GMF, PACK_CHK), lambda i: (0, i)),
    ],
    out_specs=[
        pl.BlockSpec((PACK_CHK // 4, D_MLP), lambda i: (i, 0)),
        pl.BlockSpec((PACK_CHK // 4, D_MLP), lambda i: (i, 0)),
    ],
    out_shape=[
        jax.ShapeDtypeStruct((PACK_GRID * PACK_CHK // 4, D_MLP), jnp.float32),
        jax.ShapeDtypeStruct((PACK_GRID * PACK_CHK // 4, D_MLP), jnp.float32),
    ],
    compiler_params=pltpu.CompilerParams(fuse_transposed_lhs_in_matmul=True),
)


BLK = 2048
NB = B // BLK


def _tc_body(ug, ig, um, im, ohu, ohi, w1a, w1b, b1r, w2, b2r, w3, b3r, wpg,
             wph, bpr, out):
    ug128, ig128 = ug[...], ig[...]
    g_u = sum(ohu[...][:, k:k + 1] * ug128[:, 32 * k:32 * k + 32]
              for k in range(4))
    g_i = sum(ohi[...][:, k:k + 1] * ig128[:, 32 * k:32 * k + 32]
              for k in range(4))
    g = g_u * g_i
    h = jnp.dot(um[...], w1a[...], preferred_element_type=jnp.float32)
    h = h + jnp.dot(im[...], w1b[...], preferred_element_type=jnp.float32)
    h = jnp.maximum(h + b1r[...], 0.0)
    h = jnp.maximum(
        jnp.dot(h, w2[...], preferred_element_type=jnp.float32) + b2r[...], 0.0)
    h = jnp.maximum(
        jnp.dot(h, w3[...], preferred_element_type=jnp.float32) + b3r[...], 0.0)
    p = jnp.dot(g, wpg[...], preferred_element_type=jnp.float32)
    p = p + jnp.dot(h, wph[...], preferred_element_type=jnp.float32)
    out[...] = p + bpr[...]


def _full(shape):
    return pl.BlockSpec(shape, lambda i: (0, 0))


_tc_call = pl.pallas_call(
    _tc_body,
    grid=(NB,),
    in_specs=[
        pl.BlockSpec((BLK, D_MLP), lambda i: (i, 0)),
        pl.BlockSpec((BLK, D_MLP), lambda i: (i, 0)),
        pl.BlockSpec((BLK, D_MLP), lambda i: (i, 0)),
        pl.BlockSpec((BLK, D_MLP), lambda i: (i, 0)),
        pl.BlockSpec((BLK, 4), lambda i: (i, 0)),
        pl.BlockSpec((BLK, 4), lambda i: (i, 0)),
        _full((128, 128)),
        _full((128, 128)),
        _full((1, 128)),
        _full((128, 64)),
        _full((1, 64)),
        _full((64, 32)),
        _full((1, 32)),
        _full((32, 8)),
        _full((32, 8)),
        _full((1, 8)),
    ],
    out_specs=pl.BlockSpec((BLK, 8), lambda i: (i, 0)),
    out_shape=jax.ShapeDtypeStruct((B, 8), jnp.float32),
)


def kernel(user, item, embed_user_gmf, embed_item_gmf, embed_user_mlp,
           embed_item_mlp, W1, b1, W2, b2, W3, b3, W_pred, b_pred):
    u3 = user.reshape(NW, NCH, CH)
    i3 = item.reshape(NW, NCH, CH)
    g_um, g_im = _make_sc_mlp_gather()(
        u3, i3, embed_user_mlp, embed_item_mlp)
    q = PACK_Q
    u4 = ((user >> PACK_SHIFT) * q + (user & (q - 1))).reshape(NW, NCH, CH)
    i4 = ((item >> PACK_SHIFT) * q + (item & (q - 1))).reshape(NW, NCH, CH)
    pk_u, pk_i = _pack_call(embed_user_gmf.T, embed_item_gmf.T)
    g_ug, g_ig = _make_sc_gmf_gather()(u4, i4, pk_u, pk_i)
    lanes = jnp.arange(4, dtype=jnp.int32)[None, :]
    ohu = (((user & (PACK_CHK - 1)) // q)[:, None] == lanes).astype(
        jnp.float32)
    ohi = (((item & (PACK_CHK - 1)) // q)[:, None] == lanes).astype(
        jnp.float32)
    w1t = W1.T
    w1a, w1b = w1t[:D_MLP], w1t[D_MLP:]
    wpt = W_pred.T
    wpg = jnp.broadcast_to(wpt[:D_GMF], (D_GMF, 8))
    wph = jnp.broadcast_to(wpt[D_GMF:], (D_GMF, 8))
    bpr = jnp.broadcast_to(b_pred.reshape(1, 1), (1, 8))
    p8 = _tc_call(g_ug, g_ig, g_um, g_im, ohu, ohi, w1a, w1b,
                  b1.reshape(1, -1), W2.T, b2.reshape(1, -1), W3.T,
                  b3.reshape(1, -1), wpg, wph, bpr)
    return p8[:, 0]
